# baseline (speedup 1.0000x reference)
import jax
import jax.numpy as jnp
from jax import lax
from jax.experimental import pallas as pl
from jax.experimental.pallas import tpu as pltpu
from jax.experimental.pallas import tpu_sc as plsc

B = 16384
F = 26
ROWS_PER_FIELD = 40000
NC = 2
NS = 16
NW = NC * NS
BPW = B // NW
LANES = 16
CHUNK = 128
CPF = BPW // CHUNK
NCHUNK = BPW * F // CHUNK
GPC = CHUNK // LANES


def _body(x_hbm, table_hbm, bias_hbm, out_hbm, x_v, idx_v, emb_v, out_v, bias_v, sem):
    wid = lax.axis_index("s") * NC + lax.axis_index("c")
    base = wid * BPW

    pltpu.sync_copy(x_hbm.at[:, pl.ds(base, BPW)], x_v)
    pltpu.sync_copy(bias_hbm, bias_v)

    def build(j, _):
        f = j // CPF
        c = j % CPF
        off = f * ROWS_PER_FIELD
        for g in range(GPC):
            bpos = c * CHUNK + g * LANES
            idx_v[j, pl.ds(g * LANES, LANES)] = x_v[f, pl.ds(bpos, LANES)] + off
        return 0

    lax.fori_loop(0, NCHUNK, build, 0)

    def gather(j, _):
        pltpu.async_copy(table_hbm.at[idx_v.at[j]], emb_v.at[j], sem).wait()
        return 0

    lax.fori_loop(0, NCHUNK, gather, 0)

    bias_vec = bias_v[...]

    def red(s, _):
        c = s // GPC
        col = (s % GPC) * LANES

        def add_f(f, acc):
            return acc + emb_v[f * CPF + c, pl.ds(col, LANES)]

        out_v[pl.ds(s * LANES, LANES)] = lax.fori_loop(0, F, add_f, bias_vec)
        return 0

    lax.fori_loop(0, BPW // LANES, red, 0)

    pltpu.sync_copy(out_v, out_hbm.at[pl.ds(base, BPW)])


def kernel(x, table, bias):
    xf = x.astype(jnp.int32).T
    tf = table.reshape(-1)
    bb = jnp.tile(bias.astype(jnp.float32), LANES)
    run = pl.kernel(
        _body,
        mesh=plsc.VectorSubcoreMesh(core_axis_name="c", subcore_axis_name="s"),
        out_type=jax.ShapeDtypeStruct((B,), jnp.float32),
        scratch_types=[
            pltpu.VMEM((F, BPW), jnp.int32),
            pltpu.VMEM((NCHUNK, CHUNK), jnp.int32),
            pltpu.VMEM((NCHUNK, CHUNK), jnp.float32),
            pltpu.VMEM((BPW,), jnp.float32),
            pltpu.VMEM((LANES,), jnp.float32),
            pltpu.SemaphoreType.DMA,
        ],
    )
    out = run(xf, tf, bb)
    return out.reshape(B, 1)


# speedup vs baseline: 1.6668x; 1.6668x over previous
import jax
import jax.numpy as jnp
from jax import lax
from jax.experimental import pallas as pl
from jax.experimental.pallas import tpu as pltpu
from jax.experimental.pallas import tpu_sc as plsc

B = 16384
F = 26
ROWS_PER_FIELD = 40000
NC = 2
NS = 16
NW = NC * NS
BPW = B // NW
LANES = 16
CHUNK = 128
CPF = BPW // CHUNK
NCHUNK = BPW * F // CHUNK
GPC = CHUNK // LANES
PIPE = 8


def _body(x_hbm, table_hbm, bias_hbm, out_hbm, x_v, idx_v, emb_v, out_v, bias_v, sem):
    wid = lax.axis_index("s") * NC + lax.axis_index("c")
    base = wid * BPW

    pltpu.sync_copy(x_hbm.at[:, pl.ds(base, BPW)], x_v)
    pltpu.sync_copy(bias_hbm, bias_v)

    def build(j):
        f = j // CPF
        c = j % CPF
        off = f * ROWS_PER_FIELD
        for g in range(GPC):
            bpos = c * CHUNK + g * LANES
            idx_v[j, pl.ds(g * LANES, LANES)] = x_v[f, pl.ds(bpos, LANES)] + off

    def fire(j):
        pltpu.async_copy(table_hbm.at[idx_v.at[j]], emb_v.at[j], sem)

    def drain(j):
        pltpu.make_async_copy(table_hbm.at[idx_v.at[j]], emb_v.at[j], sem).wait()

    def prologue(j, _):
        build(j)
        fire(j)
        return 0

    def steady(j, _):
        build(j)
        fire(j)
        drain(j - PIPE)
        return 0

    def epilogue(j, _):
        drain(j)
        return 0

    lax.fori_loop(0, PIPE, prologue, 0)
    lax.fori_loop(PIPE, NCHUNK, steady, 0)
    lax.fori_loop(NCHUNK - PIPE, NCHUNK, epilogue, 0)

    bias_vec = bias_v[...]

    def red(s, _):
        c = s // GPC
        col = (s % GPC) * LANES
        acc = bias_vec
        for f in range(F):
            acc = acc + emb_v[f * CPF + c, pl.ds(col, LANES)]
        out_v[pl.ds(s * LANES, LANES)] = acc
        return 0

    lax.fori_loop(0, BPW // LANES, red, 0)

    pltpu.sync_copy(out_v, out_hbm.at[pl.ds(base, BPW)])


def kernel(x, table, bias):
    xf = x.astype(jnp.int32).T
    tf = table.reshape(-1)
    bb = jnp.tile(bias.astype(jnp.float32), LANES)
    run = pl.kernel(
        _body,
        mesh=plsc.VectorSubcoreMesh(core_axis_name="c", subcore_axis_name="s"),
        out_type=jax.ShapeDtypeStruct((B,), jnp.float32),
        scratch_types=[
            pltpu.VMEM((F, BPW), jnp.int32),
            pltpu.VMEM((NCHUNK, CHUNK), jnp.int32),
            pltpu.VMEM((NCHUNK, CHUNK), jnp.float32),
            pltpu.VMEM((BPW,), jnp.float32),
            pltpu.VMEM((LANES,), jnp.float32),
            pltpu.SemaphoreType.DMA,
        ],
    )
    out = run(xf, tf, bb)
    return out.reshape(B, 1)


# speedup vs baseline: 1.8663x; 1.1196x over previous
import jax
import jax.numpy as jnp
from jax import lax
from jax.experimental import pallas as pl
from jax.experimental.pallas import tpu as pltpu
from jax.experimental.pallas import tpu_sc as plsc

B = 16384
F = 26
ROWS_PER_FIELD = 40000
NC = 2
NS = 16
NW = NC * NS
BPW = B // NW
LANES = 16
CHUNK = 128
CPF = BPW // CHUNK
NCHUNK = BPW * F // CHUNK
GPC = CHUNK // LANES
PIPE = 8


def _body(x_hbm, table_hbm, bias_hbm, out_hbm, x_v, idx_v, emb_v, out_v, bias_v, sem):
    wid = lax.axis_index("s") * NC + lax.axis_index("c")
    base = wid * BPW

    pltpu.sync_copy(x_hbm.at[:, pl.ds(base, BPW)], x_v)
    pltpu.sync_copy(bias_hbm, bias_v)

    def build(j):
        f = j // CPF
        c = j % CPF
        off = f * ROWS_PER_FIELD
        for g in range(GPC):
            bpos = c * CHUNK + g * LANES
            idx_v[j, pl.ds(g * LANES, LANES)] = x_v[f, pl.ds(bpos, LANES)] + off

    def step(j, _):
        build(j)
        pltpu.async_copy(
            table_hbm.at[idx_v.at[j]], emb_v.at[pl.ds(j * CHUNK, CHUNK)], sem
        )
        return 0

    lax.fori_loop(0, NCHUNK, step, 0)
    pltpu.make_async_copy(table_hbm.at[pl.ds(0, NCHUNK * CHUNK)], emb_v, sem).wait()

    bias_vec = bias_v[...]

    def red(s, _):
        c = s // GPC
        col = (s % GPC) * LANES
        acc = bias_vec
        for f in range(F):
            acc = acc + emb_v[pl.ds((f * CPF + c) * CHUNK + col, LANES)]
        out_v[pl.ds(s * LANES, LANES)] = acc
        return 0

    lax.fori_loop(0, BPW // LANES, red, 0)

    pltpu.sync_copy(out_v, out_hbm.at[pl.ds(base, BPW)])


def kernel(x, table, bias):
    xf = x.astype(jnp.int32).T
    tf = table.reshape(-1)
    bb = jnp.tile(bias.astype(jnp.float32), LANES)
    run = pl.kernel(
        _body,
        mesh=plsc.VectorSubcoreMesh(core_axis_name="c", subcore_axis_name="s"),
        out_type=jax.ShapeDtypeStruct((B,), jnp.float32),
        scratch_types=[
            pltpu.VMEM((F, BPW), jnp.int32),
            pltpu.VMEM((NCHUNK, CHUNK), jnp.int32),
            pltpu.VMEM((NCHUNK * CHUNK,), jnp.float32),
            pltpu.VMEM((BPW,), jnp.float32),
            pltpu.VMEM((LANES,), jnp.float32),
            pltpu.SemaphoreType.DMA,
        ],
    )
    out = run(xf, tf, bb)
    return out.reshape(B, 1)
